# baseline (speedup 1.0000x reference)
import jax
import jax.numpy as jnp
from jax.experimental import pallas as pl
from jax.experimental.pallas import tpu as pltpu

_B, _T, _N, _F = 16, 32, 68, 128
_SD, _TD, _NC, _E = 256, 256, 500, 680
_CD = 256
_TB = _T * _B
_FPP = 2
_K = _FPP * _B

_DNT = (((1,), (1,)), ((), ()))


def _dot(a, b, precision=None):
    return jnp.dot(a, b, preferred_element_type=jnp.float32, precision=precision)

_FAST = jax.lax.Precision.DEFAULT


def _dot_t(a, b):
    return jax.lax.dot_general(a, b, _DNT, preferred_element_type=jnp.float32)


def _adj_body(src_ref, dst_ref, a_ref):
    src = src_ref[...]
    dst = dst_ref[...]
    os_ = (src == jax.lax.broadcasted_iota(jnp.int32, (_E, _N), 1)).astype(
        jnp.float32
    )
    odT = (dst == jax.lax.broadcasted_iota(jnp.int32, (_N, _E), 0)).astype(
        jnp.float32
    )
    count = jax.lax.dot_general(
        odT, os_, (((1,), (0,)), ((), ())), preferred_element_type=jnp.float32
    )
    countT = jax.lax.dot_general(
        os_, odT, (((0,), (1,)), ((), ())), preferred_element_type=jnp.float32
    )
    deg_c = jnp.sum(count, axis=1, keepdims=True) + 1.0
    deg_r = jnp.sum(countT, axis=0, keepdims=True) + 1.0
    eye = (
        jax.lax.broadcasted_iota(jnp.int32, (_N, _N), 0)
        == jax.lax.broadcasted_iota(jnp.int32, (_N, _N), 1)
    ).astype(jnp.float32)
    a_ref[...] = (count + eye) * jax.lax.rsqrt(deg_c) * jax.lax.rsqrt(deg_r)


def _gcn_body(a_ref, w0_ref, b0_ref, w1_ref, b1_ref, w2_ref, b2_ref, x_ref, out_ref):
    bf16 = jnp.bfloat16
    Ab = a_ref[...].astype(bf16)
    wbs = [w0_ref[...].astype(bf16), w1_ref[...].astype(bf16), w2_ref[...].astype(bf16)]
    bvs = [b0_ref[...], b1_ref[...], b2_ref[...]]
    for j in range(_FPP):
        for i in range(_B):
            h = x_ref[i, j].astype(bf16)
            hf = None
            for wb, bv in zip(wbs, bvs):
                y = _dot(h, wb)
                g = _dot(Ab, y.astype(bf16))
                hf = jnp.maximum(g + bv, 0.0)
                h = hf.astype(bf16)
            k = j * _B + i
            out_ref[0, 0, k * _SD : (k + 1) * _SD] = jnp.mean(hf, axis=0)


def _lstm_body(
    seq_ref,
    wih0f_ref,
    whh0f_ref,
    wih0b_ref,
    whh0b_ref,
    wih1f_ref,
    whh1f_ref,
    wih1b_ref,
    whh1b_ref,
    bias0_ref,
    bias1_ref,
    clsw1_ref,
    clsb1_ref,
    clsw2_ref,
    clsb2_ref,
    out_ref,
    g0_ref,
    seq1_ref,
    g1_ref,
):
    H4 = 4 * _TD

    seq = seq_ref[...]
    g0_ref[:, 0:H4] = _dot_t(seq, wih0f_ref[...]) + bias0_ref[:, 0:H4]
    g0_ref[:, H4 : 2 * H4] = _dot_t(seq, wih0b_ref[...]) + bias0_ref[:, H4 : 2 * H4]

    def cell(g, c):
        i = jax.nn.sigmoid(g[:, 0:_TD])
        f = jax.nn.sigmoid(g[:, _TD : 2 * _TD])
        gg = jnp.tanh(g[:, 2 * _TD : 3 * _TD])
        o = jax.nn.sigmoid(g[:, 3 * _TD : 4 * _TD])
        c = f * c + i * gg
        return o * jnp.tanh(c), c

    def step0(s, carry):
        hf, cf, hb, cb = carry
        gf = g0_ref[pl.ds(s * _B, _B), 0:H4] + _dot_t(hf, whh0f_ref[...])
        gb = g0_ref[pl.ds((_T - 1 - s) * _B, _B), H4 : 2 * H4] + _dot_t(
            hb, whh0b_ref[...]
        )
        hf, cf = cell(gf, cf)
        hb, cb = cell(gb, cb)
        seq1_ref[pl.ds(s * _B, _B), 0:_TD] = hf
        seq1_ref[pl.ds((_T - 1 - s) * _B, _B), _TD : 2 * _TD] = hb
        return hf, cf, hb, cb

    z = jnp.zeros((_B, _TD), jnp.float32)
    jax.lax.fori_loop(0, _T, step0, (z, z, z, z))

    seq1 = seq1_ref[...]
    g1_ref[:, 0:H4] = _dot_t(seq1, wih1f_ref[...]) + bias1_ref[:, 0:H4]
    g1_ref[:, H4 : 2 * H4] = _dot_t(seq1, wih1b_ref[...]) + bias1_ref[:, H4 : 2 * H4]

    def step1(s, carry):
        hf, cf, hb, cb = carry
        gf = g1_ref[pl.ds(s * _B, _B), 0:H4] + _dot_t(hf, whh1f_ref[...])
        gb = g1_ref[pl.ds((_T - 1 - s) * _B, _B), H4 : 2 * H4] + _dot_t(
            hb, whh1b_ref[...]
        )
        hf, cf = cell(gf, cf)
        hb, cb = cell(gb, cb)
        return hf, cf, hb, cb

    h1f, _, h1b, _ = jax.lax.fori_loop(0, _T, step1, (z, z, z, z))

    to = jnp.concatenate([h1f, h1b], axis=1)
    h = jnp.maximum(_dot(to, clsw1_ref[...]) + clsb1_ref[...], 0.0)
    out_ref[...] = _dot(h, clsw2_ref[...]) + clsb2_ref[...]


@jax.jit
def kernel(x_temporal, edge_index, gcn_W0, gcn_b0, gcn_W1, gcn_b1, gcn_W2, gcn_b2, lstm_Wih_l0f, lstm_Whh_l0f, lstm_bih_l0f, lstm_bhh_l0f, lstm_Wih_l0b, lstm_Whh_l0b, lstm_bih_l0b, lstm_bhh_l0b, lstm_Wih_l1f, lstm_Whh_l1f, lstm_bih_l1f, lstm_bhh_l1f, lstm_Wih_l1b, lstm_Whh_l1b, lstm_bih_l1b, lstm_bhh_l1b, cls_W1, cls_b1, bn_gamma, bn_beta, bn_mean, bn_var, cls_W2, cls_b2):
    f32 = jnp.float32
    H4 = 4 * _TD

    src = edge_index[0].reshape(_E, 1)
    dst = edge_index[1].reshape(1, _E)
    A = pl.pallas_call(
        _adj_body,
        out_shape=jax.ShapeDtypeStruct((_N, _N), f32),
    )(src, dst)

    bt = [b.reshape(1, _SD) for b in (gcn_b0, gcn_b1, gcn_b2)]
    ngp = _T // _FPP
    seq = pl.pallas_call(
        _gcn_body,
        grid=(ngp,),
        in_specs=[
            pl.BlockSpec((_N, _N), lambda t: (0, 0)),
            pl.BlockSpec((_F, _SD), lambda t: (0, 0)),
            pl.BlockSpec((1, _SD), lambda t: (0, 0)),
            pl.BlockSpec((_SD, _SD), lambda t: (0, 0)),
            pl.BlockSpec((1, _SD), lambda t: (0, 0)),
            pl.BlockSpec((_SD, _SD), lambda t: (0, 0)),
            pl.BlockSpec((1, _SD), lambda t: (0, 0)),
            pl.BlockSpec((_B, _FPP, _N, _F), lambda t: (0, t, 0, 0)),
        ],
        out_specs=pl.BlockSpec((1, 1, _K * _SD), lambda t: (t, 0, 0)),
        out_shape=jax.ShapeDtypeStruct((ngp, 1, _K * _SD), f32),
    )(A, gcn_W0, bt[0], gcn_W1, bt[1], gcn_W2, bt[2], x_temporal)
    seq = seq.reshape(_TB, _SD)

    bias0 = jnp.concatenate(
        [lstm_bih_l0f + lstm_bhh_l0f, lstm_bih_l0b + lstm_bhh_l0b]
    ).reshape(1, 2 * H4)
    bias1 = jnp.concatenate(
        [lstm_bih_l1f + lstm_bhh_l1f, lstm_bih_l1b + lstm_bhh_l1b]
    ).reshape(1, 2 * H4)

    scale = bn_gamma * jax.lax.rsqrt(bn_var + 1e-5)
    w1s = cls_W1 * scale[None, :]
    b1s = ((cls_b1 - bn_mean) * scale + bn_beta).reshape(1, _CD)

    logits = pl.pallas_call(
        _lstm_body,
        out_shape=jax.ShapeDtypeStruct((_B, _NC), f32),
        scratch_shapes=[
            pltpu.VMEM((_TB, 2 * H4), f32),
            pltpu.VMEM((_TB, 2 * _TD), f32),
            pltpu.VMEM((_TB, 2 * H4), f32),
        ],
    )(
        seq,
        lstm_Wih_l0f,
        lstm_Whh_l0f,
        lstm_Wih_l0b,
        lstm_Whh_l0b,
        lstm_Wih_l1f,
        lstm_Whh_l1f,
        lstm_Wih_l1b,
        lstm_Whh_l1b,
        bias0,
        bias1,
        w1s,
        b1s,
        cls_W2,
        cls_b2.reshape(1, _NC),
    )
    return logits


# speedup vs baseline: 3.7127x; 3.7127x over previous
import jax
import jax.numpy as jnp
from jax.experimental import pallas as pl
from jax.experimental.pallas import tpu as pltpu

_B, _T, _N, _F = 16, 32, 68, 128
_SD, _TD, _NC, _E = 256, 256, 500, 680
_CD = 256
_TB = _T * _B
_FPP = 2
_K = _FPP * _B

_DNT = (((1,), (1,)), ((), ()))


def _dot(a, b, precision=None):
    return jnp.dot(a, b, preferred_element_type=jnp.float32, precision=precision)

_FAST = jax.lax.Precision.DEFAULT


def _dot_t(a, b):
    return jax.lax.dot_general(a, b, _DNT, preferred_element_type=jnp.float32)


def _adj_body(src_ref, dst_ref, a_ref):
    src = src_ref[...]
    dst = dst_ref[...]
    os_ = (src == jax.lax.broadcasted_iota(jnp.int32, (_E, _N), 1)).astype(
        jnp.float32
    )
    odT = (dst == jax.lax.broadcasted_iota(jnp.int32, (_N, _E), 0)).astype(
        jnp.float32
    )
    count = jax.lax.dot_general(
        odT, os_, (((1,), (0,)), ((), ())), preferred_element_type=jnp.float32
    )
    countT = jax.lax.dot_general(
        os_, odT, (((0,), (1,)), ((), ())), preferred_element_type=jnp.float32
    )
    deg_c = jnp.sum(count, axis=1, keepdims=True) + 1.0
    deg_r = jnp.sum(countT, axis=0, keepdims=True) + 1.0
    eye = (
        jax.lax.broadcasted_iota(jnp.int32, (_N, _N), 0)
        == jax.lax.broadcasted_iota(jnp.int32, (_N, _N), 1)
    ).astype(jnp.float32)
    a_ref[...] = (count + eye) * jax.lax.rsqrt(deg_c) * jax.lax.rsqrt(deg_r)


def _gcn_body(a_ref, w0_ref, b0_ref, w1_ref, b1_ref, w2_ref, b2_ref, x_ref, out_ref):
    A = a_ref[...]

    def layer(h, w_ref, b_ref, bt, din):
        w = w_ref[...]
        y = jnp.concatenate(
            [_dot(h[:, i * din : (i + 1) * din], w) for i in range(_K)], axis=1
        )
        return jnp.maximum(_dot(A, y) + bt, 0.0)

    h = jnp.concatenate(
        [x_ref[i, j] for j in range(_FPP) for i in range(_B)], axis=1
    )
    bts = [jnp.tile(b_ref[...], (1, _K)) for b_ref in (b0_ref, b1_ref, b2_ref)]
    h = layer(h, w0_ref, b0_ref, bts[0], _F)
    h = layer(h, w1_ref, b1_ref, bts[1], _SD)
    h = layer(h, w2_ref, b2_ref, bts[2], _SD)
    out_ref[0, 0, :] = jnp.mean(h, axis=0)


def _lstm_body(
    seq_ref,
    wih0f_ref,
    whh0f_ref,
    wih0b_ref,
    whh0b_ref,
    wih1f_ref,
    whh1f_ref,
    wih1b_ref,
    whh1b_ref,
    bias0_ref,
    bias1_ref,
    clsw1_ref,
    clsb1_ref,
    clsw2_ref,
    clsb2_ref,
    out_ref,
    g0_ref,
    seq1_ref,
    g1_ref,
):
    H4 = 4 * _TD

    seq = seq_ref[...]
    g0_ref[:, 0:H4] = _dot_t(seq, wih0f_ref[...]) + bias0_ref[:, 0:H4]
    g0_ref[:, H4 : 2 * H4] = _dot_t(seq, wih0b_ref[...]) + bias0_ref[:, H4 : 2 * H4]

    def cell(g, c):
        i = jax.nn.sigmoid(g[:, 0:_TD])
        f = jax.nn.sigmoid(g[:, _TD : 2 * _TD])
        gg = jnp.tanh(g[:, 2 * _TD : 3 * _TD])
        o = jax.nn.sigmoid(g[:, 3 * _TD : 4 * _TD])
        c = f * c + i * gg
        return o * jnp.tanh(c), c

    def step0(s, carry):
        hf, cf, hb, cb = carry
        gf = g0_ref[pl.ds(s * _B, _B), 0:H4] + _dot_t(hf, whh0f_ref[...])
        gb = g0_ref[pl.ds((_T - 1 - s) * _B, _B), H4 : 2 * H4] + _dot_t(
            hb, whh0b_ref[...]
        )
        hf, cf = cell(gf, cf)
        hb, cb = cell(gb, cb)
        seq1_ref[pl.ds(s * _B, _B), 0:_TD] = hf
        seq1_ref[pl.ds((_T - 1 - s) * _B, _B), _TD : 2 * _TD] = hb
        return hf, cf, hb, cb

    z = jnp.zeros((_B, _TD), jnp.float32)
    jax.lax.fori_loop(0, _T, step0, (z, z, z, z))

    seq1 = seq1_ref[...]
    g1_ref[:, 0:H4] = _dot_t(seq1, wih1f_ref[...]) + bias1_ref[:, 0:H4]
    g1_ref[:, H4 : 2 * H4] = _dot_t(seq1, wih1b_ref[...]) + bias1_ref[:, H4 : 2 * H4]

    def step1(s, carry):
        hf, cf, hb, cb = carry
        gf = g1_ref[pl.ds(s * _B, _B), 0:H4] + _dot_t(hf, whh1f_ref[...])
        gb = g1_ref[pl.ds((_T - 1 - s) * _B, _B), H4 : 2 * H4] + _dot_t(
            hb, whh1b_ref[...]
        )
        hf, cf = cell(gf, cf)
        hb, cb = cell(gb, cb)
        return hf, cf, hb, cb

    h1f, _, h1b, _ = jax.lax.fori_loop(0, _T, step1, (z, z, z, z))

    to = jnp.concatenate([h1f, h1b], axis=1)
    h = jnp.maximum(_dot(to, clsw1_ref[...]) + clsb1_ref[...], 0.0)
    out_ref[...] = _dot(h, clsw2_ref[...]) + clsb2_ref[...]


@jax.jit
def kernel(x_temporal, edge_index, gcn_W0, gcn_b0, gcn_W1, gcn_b1, gcn_W2, gcn_b2, lstm_Wih_l0f, lstm_Whh_l0f, lstm_bih_l0f, lstm_bhh_l0f, lstm_Wih_l0b, lstm_Whh_l0b, lstm_bih_l0b, lstm_bhh_l0b, lstm_Wih_l1f, lstm_Whh_l1f, lstm_bih_l1f, lstm_bhh_l1f, lstm_Wih_l1b, lstm_Whh_l1b, lstm_bih_l1b, lstm_bhh_l1b, cls_W1, cls_b1, bn_gamma, bn_beta, bn_mean, bn_var, cls_W2, cls_b2):
    f32 = jnp.float32
    H4 = 4 * _TD

    src = edge_index[0].reshape(_E, 1)
    dst = edge_index[1].reshape(1, _E)
    A = pl.pallas_call(
        _adj_body,
        out_shape=jax.ShapeDtypeStruct((_N, _N), f32),
    )(src, dst)

    bt = [b.reshape(1, _SD) for b in (gcn_b0, gcn_b1, gcn_b2)]
    ngp = _T // _FPP
    seq = pl.pallas_call(
        _gcn_body,
        grid=(ngp,),
        in_specs=[
            pl.BlockSpec((_N, _N), lambda t: (0, 0)),
            pl.BlockSpec((_F, _SD), lambda t: (0, 0)),
            pl.BlockSpec((1, _SD), lambda t: (0, 0)),
            pl.BlockSpec((_SD, _SD), lambda t: (0, 0)),
            pl.BlockSpec((1, _SD), lambda t: (0, 0)),
            pl.BlockSpec((_SD, _SD), lambda t: (0, 0)),
            pl.BlockSpec((1, _SD), lambda t: (0, 0)),
            pl.BlockSpec((_B, _FPP, _N, _F), lambda t: (0, t, 0, 0)),
        ],
        out_specs=pl.BlockSpec((1, 1, _K * _SD), lambda t: (t, 0, 0)),
        out_shape=jax.ShapeDtypeStruct((ngp, 1, _K * _SD), f32),
    )(A, gcn_W0, bt[0], gcn_W1, bt[1], gcn_W2, bt[2], x_temporal)
    seq = seq.reshape(_TB, _SD)

    bias0 = jnp.concatenate(
        [lstm_bih_l0f + lstm_bhh_l0f, lstm_bih_l0b + lstm_bhh_l0b]
    ).reshape(1, 2 * H4)
    bias1 = jnp.concatenate(
        [lstm_bih_l1f + lstm_bhh_l1f, lstm_bih_l1b + lstm_bhh_l1b]
    ).reshape(1, 2 * H4)

    scale = bn_gamma * jax.lax.rsqrt(bn_var + 1e-5)
    w1s = cls_W1 * scale[None, :]
    b1s = ((cls_b1 - bn_mean) * scale + bn_beta).reshape(1, _CD)

    logits = pl.pallas_call(
        _lstm_body,
        out_shape=jax.ShapeDtypeStruct((_B, _NC), f32),
        scratch_shapes=[
            pltpu.VMEM((_TB, 2 * H4), f32),
            pltpu.VMEM((_TB, 2 * _TD), f32),
            pltpu.VMEM((_TB, 2 * H4), f32),
        ],
    )(
        seq,
        lstm_Wih_l0f,
        lstm_Whh_l0f,
        lstm_Wih_l0b,
        lstm_Whh_l0b,
        lstm_Wih_l1f,
        lstm_Whh_l1f,
        lstm_Wih_l1b,
        lstm_Whh_l1b,
        bias0,
        bias1,
        w1s,
        b1s,
        cls_W2,
        cls_b2.reshape(1, _NC),
    )
    return logits
